# baseline (speedup 1.0000x reference)
import functools

import jax
import jax.numpy as jnp
from jax import lax
from jax.experimental import pallas as pl
from jax.experimental.pallas import tpu as pltpu
from jax.experimental.pallas import tpu_sc as plsc

HD = 128
N = 10000
E = 320000

NC = 2
NS = 16
NW = NC * NS
CHUNK = 128
NPAD = 10240
NBLK_A = NPAD // CHUNK
NLOC = 80
NBLK_E = NW * NLOC
EP = NBLK_E * CHUNK
SINK = N + 128
ZROWS = NPAD // NS


@functools.cache
def _sc_mesh():
    return plsc.VectorSubcoreMesh(core_axis_name="c", subcore_axis_name="s",
                                  num_cores=NC, num_subcores=NS)


def _wid():
    return lax.axis_index("s") * NC + lax.axis_index("c")


@functools.cache
def _edge_gather_call():
    @functools.partial(
        pl.kernel,
        out_type=jax.ShapeDtypeStruct((EP, HD), jnp.float32),
        mesh=_sc_mesh(),
        scratch_types=[
            pltpu.VMEM((NLOC, CHUNK), jnp.int32),
            pltpu.VMEM((NLOC, CHUNK), jnp.int32),
            pltpu.VMEM((CHUNK, HD), jnp.float32),
            pltpu.VMEM((CHUNK, HD), jnp.float32),
            pltpu.VMEM((CHUNK, HD), jnp.float32),
            pltpu.VMEM((CHUNK, HD), jnp.float32),
            pltpu.VMEM((CHUNK, HD), jnp.float32),
            pltpu.VMEM((CHUNK, HD), jnp.float32),
            pltpu.SemaphoreType.DMA,
            pltpu.SemaphoreType.DMA,
            pltpu.SemaphoreType.DMA,
            pltpu.SemaphoreType.DMA,
            pltpu.SemaphoreType.DMA,
        ],
    )
    def _edge_gather(tab_u, tab_v, idx_u, idx_v, out,
                     ixu, ixv, bu0, bu1, bv0, bv1, wb0, wb1,
                     sg0, sg1, sw0, sw1, si):
        wid = _wid()
        start = wid * NLOC
        bu = (bu0, bu1)
        bv = (bv0, bv1)
        wb = (wb0, wb1)
        sg = (sg0, sg1)
        sw = (sw0, sw1)

        du = pltpu.async_copy(idx_u.at[pl.ds(start, NLOC)], ixu, si)
        dv = pltpu.async_copy(idx_v.at[pl.ds(start, NLOC)], ixv, si)
        du.wait()
        dv.wait()

        def gather_blk(k, s):
            pltpu.async_copy(tab_u.at[ixu.at[k]], bu[s], sg[s])
            pltpu.async_copy(tab_v.at[ixv.at[k]], bv[s], sg[s])

        def wait_gather(s):
            pltpu.make_async_copy(tab_u.at[ixu.at[0]], bu[s], sg[s]).wait()
            pltpu.make_async_copy(tab_v.at[ixv.at[0]], bv[s], sg[s]).wait()

        def wait_write(s):
            pltpu.make_async_copy(wb[s], out.at[pl.ds(0, CHUNK)], sw[s]).wait()

        gather_blk(0, 0)
        gather_blk(1, 1)

        def grp(g, _):
            for s in range(2):
                k = g * 2 + s
                wait_gather(s)

                @pl.when(g >= 1)
                def _():
                    wait_write(s)
                def add_row(r, _):
                    for c in range(8):
                        sl = pl.ds(c * 16, 16)
                        wb[s][r, sl] = bu[s][r, sl] + bv[s][r, sl]
                    return 0
                lax.fori_loop(0, CHUNK, add_row, 0)

                @pl.when(g < NLOC // 2 - 1)
                def _():
                    gather_blk(k + 2, s)
                pltpu.async_copy(
                    wb[s], out.at[pl.ds((start + k) * CHUNK, CHUNK)], sw[s])
            return 0

        lax.fori_loop(0, NLOC // 2, grp, 0)
        wait_write(0)
        wait_write(1)

    return _edge_gather


@functools.cache
def _emb_gather_call():
    @functools.partial(
        pl.kernel,
        out_type=jax.ShapeDtypeStruct((NPAD, HD), jnp.float32),
        mesh=_sc_mesh(),
        scratch_types=[
            pltpu.VMEM((3, CHUNK), jnp.int32),
            pltpu.VMEM((CHUNK, HD), jnp.float32),
            pltpu.VMEM((CHUNK, HD), jnp.float32),
            pltpu.VMEM((CHUNK, HD), jnp.float32),
            pltpu.SemaphoreType.DMA,
            pltpu.SemaphoreType.DMA,
            pltpu.SemaphoreType.DMA,
        ],
    )
    def _emb_gather(tab_hbm, idx_hbm, out_hbm, ix, rb0, rb1, rb2,
                    si, sg, sw):
        wid = _wid()
        nloc = (NBLK_A - wid + NW - 1) // NW
        rb = (rb0, rb1, rb2)

        for k in range(3):
            @pl.when(k < nloc)
            def _():
                b = wid + k * NW
                pltpu.async_copy(idx_hbm.at[pl.ds(b, 1)],
                                 ix.at[pl.ds(k, 1)], si)
        for k in range(3):
            @pl.when(k < nloc)
            def _():
                pltpu.make_async_copy(idx_hbm.at[pl.ds(0, 1)],
                                      ix.at[pl.ds(0, 1)], si).wait()
        for k in range(3):
            @pl.when(k < nloc)
            def _():
                pltpu.async_copy(tab_hbm.at[ix.at[k]], rb[k], sg)
        for k in range(3):
            @pl.when(k < nloc)
            def _():
                pltpu.make_async_copy(tab_hbm.at[ix.at[0]], rb[0], sg).wait()
        for k in range(3):
            @pl.when(k < nloc)
            def _():
                b = wid + k * NW
                pltpu.async_copy(rb[k], out_hbm.at[pl.ds(b * CHUNK, CHUNK)],
                                 sw)
        for k in range(3):
            @pl.when(k < nloc)
            def _():
                pltpu.make_async_copy(rb[0],
                                      out_hbm.at[pl.ds(0, CHUNK)], sw).wait()

    return _emb_gather


@functools.cache
def _edge_scatter_call():
    @functools.partial(
        pl.kernel,
        out_type=jax.ShapeDtypeStruct((NC, NPAD, HD), jnp.float32),
        mesh=_sc_mesh(),
        scratch_types=[
            pltpu.VMEM((CHUNK,), jnp.int32),
            pltpu.VMEM((CHUNK,), jnp.int32),
            pltpu.VMEM((CHUNK, HD), jnp.float32),
            pltpu.VMEM((CHUNK, HD), jnp.float32),
            pltpu.VMEM_SHARED((NPAD, HD), jnp.float32),
            pltpu.SemaphoreType.DMA,
            pltpu.SemaphoreType.DMA,
            pltpu.SemaphoreType.DMA,
            pltpu.SemaphoreType.DMA,
            pltpu.SemaphoreType.DMA,
        ],
    )
    def _edge_scatter(m_hbm, idx_hbm, zeros_hbm, out_hbm,
                      ix0, ix1, mb0, mb1, acc, sm0, sm1, ss0, ss1, si):
        cid = lax.axis_index("c")
        sid = lax.axis_index("s")
        wid = sid * NC + cid
        start = wid * NLOC
        ix = (ix0, ix1)
        mb = (mb0, mb1)
        sm = (sm0, sm1)
        ss = (ss0, ss1)

        dz = pltpu.async_copy(zeros_hbm.at[pl.ds(0, CHUNK)], mb0, si)
        dz.wait()
        for j in range(ZROWS // CHUNK):
            o = sid * ZROWS + j * CHUNK
            pltpu.async_copy(mb0, acc.at[pl.ds(o, CHUNK)], si)
        for j in range(ZROWS // CHUNK):
            pltpu.make_async_copy(mb0, acc.at[pl.ds(0, CHUNK)], si).wait()
        plsc.subcore_barrier()

        def load_blk(k, s):
            base = (start + k) * CHUNK
            pltpu.async_copy(idx_hbm.at[pl.ds(base, CHUNK)], ix[s], sm[s])
            pltpu.async_copy(m_hbm.at[pl.ds(base, CHUNK)], mb[s], sm[s])

        def wait_load(s):
            pltpu.make_async_copy(idx_hbm.at[pl.ds(0, CHUNK)], ix[s],
                                  sm[s]).wait()
            pltpu.make_async_copy(m_hbm.at[pl.ds(0, CHUNK)], mb[s],
                                  sm[s]).wait()

        def wait_scat(s):
            pltpu.make_async_copy(mb[s], acc.at[ix[s]], ss[s]).wait()

        load_blk(0, 0)
        load_blk(1, 1)

        def grp(g, _):
            for s in range(2):
                k = g * 2 + s
                wait_load(s)
                pltpu.async_copy(mb[s], acc.at[ix[s]], ss[s], add=True)

                @pl.when(g < NLOC // 2 - 1)
                def _():
                    wait_scat(s)
                    load_blk(k + 2, s)
            return 0

        lax.fori_loop(0, NLOC // 2, grp, 0)
        wait_scat(0)
        wait_scat(1)
        plsc.subcore_barrier()

        for j in range(ZROWS // CHUNK):
            s = j % 2
            o = sid * ZROWS + j * CHUNK
            if j >= 2:
                pltpu.make_async_copy(mb[s], out_hbm.at[cid, pl.ds(0, CHUNK)],
                                      ss[s]).wait()
            d = pltpu.async_copy(acc.at[pl.ds(o, CHUNK)], mb[s], sm[s])
            d.wait()
            pltpu.async_copy(mb[s], out_hbm.at[cid, pl.ds(o, CHUNK)], ss[s])
        for s in range(2):
            pltpu.make_async_copy(mb[s], out_hbm.at[cid, pl.ds(0, CHUNK)],
                                  ss[s]).wait()

    return _edge_scatter


def _silu(x):
    return x * jax.nn.sigmoid(x)


def _mlp_body(pre, dis, w1c, b1, W2, b2, W3, b3, out):
    x = pre[...] + dis[...] * w1c[...] + b1[...]
    x = _silu(x)
    x = jnp.dot(x, W2[...], preferred_element_type=jnp.float32) + b2[...]
    x = _silu(x)
    out[...] = jnp.dot(x, W3[...], preferred_element_type=jnp.float32) + b3[...]


_BE = 512


def _edge_mlp(pre, dis, w1c, b1, W2, b2, W3, b3):
    full = lambda i: (0, 0)
    return pl.pallas_call(
        _mlp_body,
        grid=(EP // _BE,),
        in_specs=[
            pl.BlockSpec((_BE, HD), lambda i: (i, 0)),
            pl.BlockSpec((_BE, 1), lambda i: (i, 0)),
            pl.BlockSpec((1, HD), full),
            pl.BlockSpec((1, HD), full),
            pl.BlockSpec((HD, HD), full),
            pl.BlockSpec((1, HD), full),
            pl.BlockSpec((HD, HD), full),
            pl.BlockSpec((1, HD), full),
        ],
        out_specs=pl.BlockSpec((_BE, HD), lambda i: (i, 0)),
        out_shape=jax.ShapeDtypeStruct((EP, HD), jnp.float32),
    )(pre, dis, w1c, b1, W2, b2, W3, b3)


_BN = 2048


def _upd_body(nout, h, a1, a2, W1h, W1a, W1b, b1, W2, b2, *rest):
    nexts = rest[:2 * nout]
    outs = rest[2 * nout:]
    href = h[...]
    x = (jnp.dot(href, W1h[...], preferred_element_type=jnp.float32)
         + jnp.dot(a1[0] + a1[1], W1a[...], preferred_element_type=jnp.float32)
         + jnp.dot(a2[0] + a2[1], W1b[...], preferred_element_type=jnp.float32)
         + b1[...])
    x = _silu(x)
    hn = href + jnp.dot(x, W2[...], preferred_element_type=jnp.float32) + b2[...]
    outs[0][...] = hn
    for k in range(nout):
        W, b = nexts[2 * k], nexts[2 * k + 1]
        outs[k + 1][...] = (jnp.dot(hn, W[...], preferred_element_type=jnp.float32)
                            + b[...])


def _atom_update(h, a1, a2, p, next_mats):
    full = lambda i: (0, 0)
    nout = len(next_mats)
    in_specs = [
        pl.BlockSpec((_BN, HD), lambda i: (i, 0)),
        pl.BlockSpec((NC, _BN, HD), lambda i: (0, i, 0)),
        pl.BlockSpec((NC, _BN, HD), lambda i: (0, i, 0)),
        pl.BlockSpec((HD, HD), full),
        pl.BlockSpec((HD, HD), full),
        pl.BlockSpec((HD, HD), full),
        pl.BlockSpec((1, HD), full),
        pl.BlockSpec((HD, HD), full),
        pl.BlockSpec((1, HD), full),
    ]
    args = [h, a1, a2, p['W1'][:HD], p['W1'][HD:2 * HD], p['W1'][2 * HD:],
            p['b1'][None], p['W2'], p['b2'][None]]
    out_shapes = [jax.ShapeDtypeStruct((NPAD, HD), jnp.float32)]
    out_specs = [pl.BlockSpec((_BN, HD), lambda i: (i, 0))]
    for W, b in next_mats:
        K = W.shape[1]
        in_specs += [pl.BlockSpec((HD, K), full), pl.BlockSpec((1, K), full)]
        args += [W, b]
        out_shapes.append(jax.ShapeDtypeStruct((NPAD, K), jnp.float32))
        out_specs.append(pl.BlockSpec((_BN, K), lambda i: (i, 0)))
    return pl.pallas_call(
        functools.partial(_upd_body, nout),
        grid=(NPAD // _BN,),
        in_specs=in_specs,
        out_specs=out_specs,
        out_shape=out_shapes,
    )(*args)


def _pre_body(h, Wa, Wb, Wc, Wd, oa, ob, oc, od):
    href = h[...]
    oa[...] = jnp.dot(href, Wa[...], preferred_element_type=jnp.float32)
    ob[...] = jnp.dot(href, Wb[...], preferred_element_type=jnp.float32)
    oc[...] = jnp.dot(href, Wc[...], preferred_element_type=jnp.float32)
    od[...] = jnp.dot(href, Wd[...], preferred_element_type=jnp.float32)


def _precompute_tables(h, p1, p2):
    full = lambda i: (0, 0)
    return pl.pallas_call(
        _pre_body,
        grid=(NPAD // _BN,),
        in_specs=[pl.BlockSpec((_BN, HD), lambda i: (i, 0))] +
                 [pl.BlockSpec((HD, HD), full)] * 4,
        out_specs=[pl.BlockSpec((_BN, HD), lambda i: (i, 0))] * 4,
        out_shape=[jax.ShapeDtypeStruct((NPAD, HD), jnp.float32)] * 4,
    )(h, p1['W1'][:HD], p1['W1'][HD:2 * HD], p2['W1'][:HD], p2['W1'][HD:2 * HD])


def _round(h, tabs, dis1, dis2, id1u, id1v, id2u, id2v, p1, p2, pupd,
           zeros, next_mats):
    g1u, g1v, g2u, g2v = tabs
    pre1 = _edge_gather_call()(g1u, g1v, id1u, id1v)
    pre2 = _edge_gather_call()(g2u, g2v, id2u, id2v)
    m1 = _edge_mlp(pre1, dis1, p1['W1'][2 * HD:], p1['b1'][None],
                   p1['W2'], p1['b2'][None], p1['W3'], p1['b3'][None])
    m2 = _edge_mlp(pre2, dis2, p2['W1'][2 * HD:], p2['b1'][None],
                   p2['W2'], p2['b2'][None], p2['W3'], p2['b3'][None])
    a1 = _edge_scatter_call()(m1, id1v.reshape(EP), zeros)
    a2 = _edge_scatter_call()(m2, id2v.reshape(EP), zeros)
    return _atom_update(h, a1, a2, pupd, next_mats)


def kernel(atom_num, dis1, dis2, id1u, id1v, id2u, id2v, params):
    p = params
    i32 = jnp.int32

    def pad_idx(x, fill):
        return jnp.pad(x.astype(i32), (0, EP - E),
                       constant_values=fill).reshape(NBLK_E, CHUNK)

    id1u_, id2u_ = pad_idx(id1u, 0), pad_idx(id2u, 0)
    id1v_, id2v_ = pad_idx(id1v, SINK), pad_idx(id2v, SINK)
    an = jnp.pad(atom_num.astype(i32), (0, NPAD - N)).reshape(NBLK_A, CHUNK)
    dis1 = jnp.pad(dis1, (0, EP - E))[:, None]
    dis2 = jnp.pad(dis2, (0, EP - E))[:, None]
    zeros = jnp.zeros((NPAD, HD), jnp.float32)

    h = _emb_gather_call()(p['atom_emb'], an)
    tabs1 = _precompute_tables(h, p['edge1'], p['edge2'])
    h2, g1u, g1v, g2u, g2v = _round(
        h, tabs1, dis1, dis2, id1u_, id1v_, id2u_, id2v_,
        p['edge1'], p['edge2'], p['upd1'], zeros,
        [(p['uedge1']['W1'][:HD], jnp.zeros((1, HD), jnp.float32)),
         (p['uedge1']['W1'][HD:2 * HD], jnp.zeros((1, HD), jnp.float32)),
         (p['uedge2']['W1'][:HD], jnp.zeros((1, HD), jnp.float32)),
         (p['uedge2']['W1'][HD:2 * HD], jnp.zeros((1, HD), jnp.float32))])
    (delta,) = _round(
        h2, (g1u, g1v, g2u, g2v), dis1, dis2, id1u_, id1v_, id2u_, id2v_,
        p['uedge1'], p['uedge2'], p['upd2'], zeros,
        [(p['Wout'], p['bout'][None])])[1:]
    return delta[:N]


# speedup vs baseline: 1.1289x; 1.1289x over previous
import functools

import jax
import jax.numpy as jnp
from jax import lax
from jax.experimental import pallas as pl
from jax.experimental.pallas import tpu as pltpu
from jax.experimental.pallas import tpu_sc as plsc

HD = 128
N = 10000
E = 320000

NC = 2
NS = 16
NW = NC * NS
CHUNK = 128
NPAD = 10240
NBLK_A = NPAD // CHUNK
NLOC = 80
NBLK_E = NW * NLOC
EP = NBLK_E * CHUNK
SINK = N + 128
ZROWS = NPAD // NS
HDH = HD // 2


@functools.cache
def _sc_mesh():
    return plsc.VectorSubcoreMesh(core_axis_name="c", subcore_axis_name="s",
                                  num_cores=NC, num_subcores=NS)


def _wid():
    return lax.axis_index("s") * NC + lax.axis_index("c")


@functools.cache
def _edge_gather_call():
    @functools.partial(
        pl.kernel,
        out_type=[jax.ShapeDtypeStruct((EP, HD), jnp.float32),
                  jax.ShapeDtypeStruct((EP, HD), jnp.float32)],
        mesh=_sc_mesh(),
        scratch_types=[
            pltpu.VMEM((NLOC, CHUNK), jnp.int32),
            pltpu.VMEM((NLOC, CHUNK), jnp.int32),
            pltpu.VMEM((CHUNK, HD), jnp.float32),
            pltpu.VMEM((CHUNK, HD), jnp.float32),
            pltpu.VMEM((CHUNK, HD), jnp.float32),
            pltpu.VMEM((CHUNK, HD), jnp.float32),
            pltpu.SemaphoreType.DMA,
            pltpu.SemaphoreType.DMA,
            pltpu.SemaphoreType.DMA,
            pltpu.SemaphoreType.DMA,
            pltpu.SemaphoreType.DMA,
        ],
    )
    def _edge_gather(tab_u, tab_v, idx_u, idx_v, out_u, out_v,
                     ixu, ixv, bu0, bu1, bv0, bv1,
                     sg0, sg1, sw0, sw1, si):
        wid = _wid()
        start = wid * NLOC
        bu = (bu0, bu1)
        bv = (bv0, bv1)
        sg = (sg0, sg1)
        sw = (sw0, sw1)

        du = pltpu.async_copy(idx_u.at[pl.ds(start, NLOC)], ixu, si)
        dv = pltpu.async_copy(idx_v.at[pl.ds(start, NLOC)], ixv, si)
        du.wait()
        dv.wait()

        def gather_blk(k, s):
            pltpu.async_copy(tab_u.at[ixu.at[k]], bu[s], sg[s])
            pltpu.async_copy(tab_v.at[ixv.at[k]], bv[s], sg[s])

        def wait_gather(s):
            pltpu.make_async_copy(tab_u.at[ixu.at[0]], bu[s], sg[s]).wait()
            pltpu.make_async_copy(tab_v.at[ixv.at[0]], bv[s], sg[s]).wait()

        def wait_write(s):
            pltpu.make_async_copy(bu[s], out_u.at[pl.ds(0, CHUNK)],
                                  sw[s]).wait()
            pltpu.make_async_copy(bv[s], out_v.at[pl.ds(0, CHUNK)],
                                  sw[s]).wait()

        gather_blk(0, 0)
        gather_blk(1, 1)

        def grp(g, _):
            for s in range(2):
                k = g * 2 + s
                rows = pl.ds((start + k) * CHUNK, CHUNK)
                wait_gather(s)
                pltpu.async_copy(bu[s], out_u.at[rows], sw[s])
                pltpu.async_copy(bv[s], out_v.at[rows], sw[s])

                @pl.when(g < NLOC // 2 - 1)
                def _():
                    wait_write(s)
                    gather_blk(k + 2, s)
            return 0

        lax.fori_loop(0, NLOC // 2, grp, 0)
        wait_write(0)
        wait_write(1)

    return _edge_gather


@functools.cache
def _emb_gather_call():
    @functools.partial(
        pl.kernel,
        out_type=jax.ShapeDtypeStruct((NPAD, HD), jnp.float32),
        mesh=_sc_mesh(),
        scratch_types=[
            pltpu.VMEM((3, CHUNK), jnp.int32),
            pltpu.VMEM((CHUNK, HD), jnp.float32),
            pltpu.VMEM((CHUNK, HD), jnp.float32),
            pltpu.VMEM((CHUNK, HD), jnp.float32),
            pltpu.SemaphoreType.DMA,
            pltpu.SemaphoreType.DMA,
            pltpu.SemaphoreType.DMA,
        ],
    )
    def _emb_gather(tab_hbm, idx_hbm, out_hbm, ix, rb0, rb1, rb2,
                    si, sg, sw):
        wid = _wid()
        nloc = (NBLK_A - wid + NW - 1) // NW
        rb = (rb0, rb1, rb2)

        for k in range(3):
            @pl.when(k < nloc)
            def _():
                b = wid + k * NW
                pltpu.async_copy(idx_hbm.at[pl.ds(b, 1)],
                                 ix.at[pl.ds(k, 1)], si)
        for k in range(3):
            @pl.when(k < nloc)
            def _():
                pltpu.make_async_copy(idx_hbm.at[pl.ds(0, 1)],
                                      ix.at[pl.ds(0, 1)], si).wait()
        for k in range(3):
            @pl.when(k < nloc)
            def _():
                pltpu.async_copy(tab_hbm.at[ix.at[k]], rb[k], sg)
        for k in range(3):
            @pl.when(k < nloc)
            def _():
                pltpu.make_async_copy(tab_hbm.at[ix.at[0]], rb[0], sg).wait()
        for k in range(3):
            @pl.when(k < nloc)
            def _():
                b = wid + k * NW
                pltpu.async_copy(rb[k], out_hbm.at[pl.ds(b * CHUNK, CHUNK)],
                                 sw)
        for k in range(3):
            @pl.when(k < nloc)
            def _():
                pltpu.make_async_copy(rb[0],
                                      out_hbm.at[pl.ds(0, CHUNK)], sw).wait()

    return _emb_gather


@functools.cache
def _edge_scatter_call():
    @functools.partial(
        pl.kernel,
        out_type=jax.ShapeDtypeStruct((NC, NPAD, HD), jnp.float32),
        mesh=_sc_mesh(),
        scratch_types=[
            pltpu.VMEM((CHUNK,), jnp.int32),
            pltpu.VMEM((CHUNK,), jnp.int32),
            pltpu.VMEM((CHUNK, HD), jnp.float32),
            pltpu.VMEM((CHUNK, HD), jnp.float32),
            pltpu.VMEM_SHARED((NPAD, HD), jnp.float32),
            pltpu.SemaphoreType.DMA,
            pltpu.SemaphoreType.DMA,
            pltpu.SemaphoreType.DMA,
            pltpu.SemaphoreType.DMA,
            pltpu.SemaphoreType.DMA,
        ],
    )
    def _edge_scatter(m_hbm, idx_hbm, zeros_hbm, out_hbm,
                      ix0, ix1, mb0, mb1, acc, sm0, sm1, ss0, ss1, si):
        cid = lax.axis_index("c")
        sid = lax.axis_index("s")
        wid = sid * NC + cid
        start = wid * NLOC
        ix = (ix0, ix1)
        mb = (mb0, mb1)
        sm = (sm0, sm1)
        ss = (ss0, ss1)

        dz = pltpu.async_copy(zeros_hbm.at[pl.ds(0, CHUNK)], mb0, si)
        dz.wait()
        for j in range(ZROWS // CHUNK):
            o = sid * ZROWS + j * CHUNK
            pltpu.async_copy(mb0, acc.at[pl.ds(o, CHUNK)], si)
        for j in range(ZROWS // CHUNK):
            pltpu.make_async_copy(mb0, acc.at[pl.ds(0, CHUNK)], si).wait()
        plsc.subcore_barrier()

        def load_blk(k, s):
            base = (start + k) * CHUNK
            pltpu.async_copy(idx_hbm.at[pl.ds(base, CHUNK)], ix[s], sm[s])
            pltpu.async_copy(m_hbm.at[pl.ds(base, CHUNK)], mb[s], sm[s])

        def wait_load(s):
            pltpu.make_async_copy(idx_hbm.at[pl.ds(0, CHUNK)], ix[s],
                                  sm[s]).wait()
            pltpu.make_async_copy(m_hbm.at[pl.ds(0, CHUNK)], mb[s],
                                  sm[s]).wait()

        def wait_scat(s):
            pltpu.make_async_copy(mb[s], acc.at[ix[s]], ss[s]).wait()

        load_blk(0, 0)
        load_blk(1, 1)

        def grp(g, _):
            for s in range(2):
                k = g * 2 + s
                wait_load(s)
                pltpu.async_copy(mb[s], acc.at[ix[s]], ss[s], add=True)

                @pl.when(g < NLOC // 2 - 1)
                def _():
                    wait_scat(s)
                    load_blk(k + 2, s)
            return 0

        lax.fori_loop(0, NLOC // 2, grp, 0)
        wait_scat(0)
        wait_scat(1)
        plsc.subcore_barrier()

        for j in range(ZROWS // CHUNK):
            s = j % 2
            o = sid * ZROWS + j * CHUNK
            if j >= 2:
                pltpu.make_async_copy(mb[s], out_hbm.at[cid, pl.ds(0, CHUNK)],
                                      ss[s]).wait()
            d = pltpu.async_copy(acc.at[pl.ds(o, CHUNK)], mb[s], sm[s])
            d.wait()
            pltpu.async_copy(mb[s], out_hbm.at[cid, pl.ds(o, CHUNK)], ss[s])
        for s in range(2):
            pltpu.make_async_copy(mb[s], out_hbm.at[cid, pl.ds(0, CHUNK)],
                                  ss[s]).wait()

    return _edge_scatter


def _silu(x):
    return x * jax.nn.sigmoid(x)


def _mlp_body(ru, rv, dis, w1c, b1, W2, b2, W3, b3, out):
    x = (ru[...].astype(jnp.float32) + rv[...].astype(jnp.float32)
         + dis[...] * w1c[...] + b1[...])
    x = _silu(x).astype(jnp.bfloat16)
    x = jnp.dot(x, W2[...], preferred_element_type=jnp.float32) + b2[...]
    x = _silu(x).astype(jnp.bfloat16)
    out[...] = jnp.dot(x, W3[...], preferred_element_type=jnp.float32) + b3[...]


_BE = 2048


def _edge_mlp(ru, rv, dis, w1c, b1, W2, b2, W3, b3):
    full = lambda i: (0, 0)
    return pl.pallas_call(
        _mlp_body,
        grid=(EP // _BE,),
        in_specs=[
            pl.BlockSpec((_BE, HD), lambda i: (i, 0)),
            pl.BlockSpec((_BE, HD), lambda i: (i, 0)),
            pl.BlockSpec((_BE, 1), lambda i: (i, 0)),
            pl.BlockSpec((1, HD), full),
            pl.BlockSpec((1, HD), full),
            pl.BlockSpec((HD, HD), full),
            pl.BlockSpec((1, HD), full),
            pl.BlockSpec((HD, HD), full),
            pl.BlockSpec((1, HD), full),
        ],
        out_specs=pl.BlockSpec((_BE, HD), lambda i: (i, 0)),
        out_shape=jax.ShapeDtypeStruct((EP, HD), jnp.float32),
    )(ru, rv, dis, w1c, b1, W2, b2, W3, b3)


_BN = 2048


def _upd_body(nout, h, a1, a2, W1h, W1a, W1b, b1, W2, b2, *rest):
    nexts = rest[:2 * nout]
    outs = rest[2 * nout:]
    href = h[...]
    x = (jnp.dot(href, W1h[...], preferred_element_type=jnp.float32)
         + jnp.dot(a1[0] + a1[1], W1a[...], preferred_element_type=jnp.float32)
         + jnp.dot(a2[0] + a2[1], W1b[...], preferred_element_type=jnp.float32)
         + b1[...])
    x = _silu(x)
    hn = href + jnp.dot(x, W2[...], preferred_element_type=jnp.float32) + b2[...]
    outs[0][...] = hn
    for k in range(nout):
        W, b = nexts[2 * k], nexts[2 * k + 1]
        o = outs[k + 1]
        o[...] = (jnp.dot(hn, W[...], preferred_element_type=jnp.float32)
                  + b[...]).astype(o.dtype)


def _atom_update(h, a1, a2, p, next_mats):
    full = lambda i: (0, 0)
    nout = len(next_mats)
    in_specs = [
        pl.BlockSpec((_BN, HD), lambda i: (i, 0)),
        pl.BlockSpec((NC, _BN, HD), lambda i: (0, i, 0)),
        pl.BlockSpec((NC, _BN, HD), lambda i: (0, i, 0)),
        pl.BlockSpec((HD, HD), full),
        pl.BlockSpec((HD, HD), full),
        pl.BlockSpec((HD, HD), full),
        pl.BlockSpec((1, HD), full),
        pl.BlockSpec((HD, HD), full),
        pl.BlockSpec((1, HD), full),
    ]
    args = [h, a1, a2, p['W1'][:HD], p['W1'][HD:2 * HD], p['W1'][2 * HD:],
            p['b1'][None], p['W2'], p['b2'][None]]
    out_shapes = [jax.ShapeDtypeStruct((NPAD, HD), jnp.float32)]
    out_specs = [pl.BlockSpec((_BN, HD), lambda i: (i, 0))]
    for W, b in next_mats:
        K = W.shape[1]
        in_specs += [pl.BlockSpec((HD, K), full), pl.BlockSpec((1, K), full)]
        args += [W, b]
        out_shapes.append(jax.ShapeDtypeStruct((NPAD, K), jnp.float32))
        out_specs.append(pl.BlockSpec((_BN, K), lambda i: (i, 0)))
    return pl.pallas_call(
        functools.partial(_upd_body, nout),
        grid=(NPAD // _BN,),
        in_specs=in_specs,
        out_specs=out_specs,
        out_shape=out_shapes,
    )(*args)


def _pre_body(h, Wa, Wb, Wc, Wd, oa, ob, oc, od):
    href = h[...]
    for W, o in ((Wa, oa), (Wb, ob), (Wc, oc), (Wd, od)):
        o[...] = jnp.dot(href, W[...],
                         preferred_element_type=jnp.float32).astype(o.dtype)


def _precompute_tables(h, p1, p2):
    full = lambda i: (0, 0)
    return pl.pallas_call(
        _pre_body,
        grid=(NPAD // _BN,),
        in_specs=[pl.BlockSpec((_BN, HD), lambda i: (i, 0))] +
                 [pl.BlockSpec((HD, HD), full)] * 4,
        out_specs=[pl.BlockSpec((_BN, HD), lambda i: (i, 0))] * 4,
        out_shape=[jax.ShapeDtypeStruct((NPAD, HD), jnp.float32)] * 4,
    )(h, p1['W1'][:HD], p1['W1'][HD:2 * HD], p2['W1'][:HD], p2['W1'][HD:2 * HD])


def _round(h, tabs, dis1, dis2, id1u, id1v, id2u, id2v, p1, p2, pupd,
           zeros, next_mats):
    g1u, g1v, g2u, g2v = tabs
    r1u, r1v = _edge_gather_call()(g1u, g1v, id1u, id1v)
    r2u, r2v = _edge_gather_call()(g2u, g2v, id2u, id2v)
    bf = jnp.bfloat16
    m1 = _edge_mlp(r1u, r1v, dis1, p1['W1'][2 * HD:], p1['b1'][None],
                   p1['W2'].astype(bf), p1['b2'][None],
                   p1['W3'].astype(bf), p1['b3'][None])
    m2 = _edge_mlp(r2u, r2v, dis2, p2['W1'][2 * HD:], p2['b1'][None],
                   p2['W2'].astype(bf), p2['b2'][None],
                   p2['W3'].astype(bf), p2['b3'][None])
    a1 = _edge_scatter_call()(m1, id1v.reshape(EP), zeros)
    a2 = _edge_scatter_call()(m2, id2v.reshape(EP), zeros)
    return _atom_update(h, a1, a2, pupd, next_mats)


def kernel(atom_num, dis1, dis2, id1u, id1v, id2u, id2v, params):
    p = params
    i32 = jnp.int32

    def pad_idx(x, fill):
        return jnp.pad(x.astype(i32), (0, EP - E),
                       constant_values=fill).reshape(NBLK_E, CHUNK)

    id1u_, id2u_ = pad_idx(id1u, 0), pad_idx(id2u, 0)
    id1v_, id2v_ = pad_idx(id1v, SINK), pad_idx(id2v, SINK)
    an = jnp.pad(atom_num.astype(i32), (0, NPAD - N)).reshape(NBLK_A, CHUNK)
    dis1 = jnp.pad(dis1, (0, EP - E))[:, None]
    dis2 = jnp.pad(dis2, (0, EP - E))[:, None]
    zeros = jnp.zeros((NPAD, HD), jnp.float32)

    h = _emb_gather_call()(p['atom_emb'], an)
    tabs1 = _precompute_tables(h, p['edge1'], p['edge2'])
    h2, g1u, g1v, g2u, g2v = _round(
        h, tabs1, dis1, dis2, id1u_, id1v_, id2u_, id2v_,
        p['edge1'], p['edge2'], p['upd1'], zeros,
        [(p['uedge1']['W1'][:HD], jnp.zeros((1, HD), jnp.float32)),
         (p['uedge1']['W1'][HD:2 * HD], jnp.zeros((1, HD), jnp.float32)),
         (p['uedge2']['W1'][:HD], jnp.zeros((1, HD), jnp.float32)),
         (p['uedge2']['W1'][HD:2 * HD], jnp.zeros((1, HD), jnp.float32))])
    (delta,) = _round(
        h2, (g1u, g1v, g2u, g2v), dis1, dis2, id1u_, id1v_, id2u_, id2v_,
        p['uedge1'], p['uedge2'], p['upd2'], zeros,
        [(p['Wout'], p['bout'][None])])[1:]
    return delta[:N]
